# baseline (speedup 1.0000x reference)
import jax
import jax.numpy as jnp
from jax.experimental import pallas as pl

B = 16384
D_IN = 1024
D_HID = 512
D_OUT = 512

BM = 1024


def _mlp_kernel(xa_ref, xb_ref, w1_ref, b1_ref, w2_ref, b2_ref, o_ref):
    w1 = w1_ref[:]
    w2 = w2_ref[:]
    ha = jnp.dot(xa_ref[:], w1, preferred_element_type=jnp.float32)
    ha = jnp.maximum(ha + b1_ref[:], 0.0)
    o_ref[:BM, :] = jnp.dot(ha, w2, preferred_element_type=jnp.float32) + b2_ref[:]
    hb = jnp.dot(xb_ref[:], w1, preferred_element_type=jnp.float32)
    hb = jnp.maximum(hb + b1_ref[:], 0.0)
    o_ref[BM:, :] = jnp.dot(hb, w2, preferred_element_type=jnp.float32) + b2_ref[:]


@jax.jit
def kernel(image, W1, b1, W2, b2):
    b1r = b1.reshape(1, D_HID)
    b2r = b2.reshape(1, D_OUT)
    grid = (B // (2 * BM),)
    return pl.pallas_call(
        _mlp_kernel,
        grid=grid,
        in_specs=[
            pl.BlockSpec((BM, D_IN), lambda i: (2 * i, 0)),
            pl.BlockSpec((BM, D_IN), lambda i: (2 * i + 1, 0)),
            pl.BlockSpec((D_IN, D_HID), lambda i: (0, 0)),
            pl.BlockSpec((1, D_HID), lambda i: (0, 0)),
            pl.BlockSpec((D_HID, D_OUT), lambda i: (0, 0)),
            pl.BlockSpec((1, D_OUT), lambda i: (0, 0)),
        ],
        out_specs=pl.BlockSpec((2 * BM, D_OUT), lambda i: (i, 0)),
        out_shape=jax.ShapeDtypeStruct((B, D_OUT), jnp.float32),
    )(image, image, W1, b1r, W2, b2r)


# speedup vs baseline: 1.0477x; 1.0477x over previous
import jax
import jax.numpy as jnp
from jax.experimental import pallas as pl
from jax.experimental.pallas import tpu as pltpu

B = 16384
D_IN = 1024
D_HID = 512
D_OUT = 512

CH = 1024
NCHUNK = B // CH
NBUF = 3


def _mlp_kernel(x_hbm, w1_ref, b1_ref, w2_ref, b2_ref, o_hbm,
                x_bufs, o_bufs, in_sems, out_sems):
    w1 = w1_ref[:]
    b1 = b1_ref[:]
    w2 = w2_ref[:]
    b2 = b2_ref[:]

    def in_copy(i, slot):
        return pltpu.make_async_copy(
            x_hbm.at[pl.ds(i * CH, CH), :], x_bufs.at[slot], in_sems.at[slot])

    def out_copy(i, slot):
        return pltpu.make_async_copy(
            o_bufs.at[slot], o_hbm.at[pl.ds(i * CH, CH), :], out_sems.at[slot])

    for k in range(NBUF):
        in_copy(k, k).start()

    def body(i, _):
        slot = jax.lax.rem(i, NBUF)
        in_copy(i, slot).wait()

        h = jnp.dot(x_bufs[slot], w1, preferred_element_type=jnp.float32)
        h = jnp.maximum(h + b1, 0.0)
        o = jnp.dot(h, w2, preferred_element_type=jnp.float32) + b2

        @pl.when(i >= NBUF)
        def _():
            out_copy(i - NBUF, slot).wait()

        o_bufs[slot] = o
        out_copy(i, slot).start()

        @pl.when(i + NBUF < NCHUNK)
        def _():
            in_copy(i + NBUF, slot).start()

        return 0

    jax.lax.fori_loop(0, NCHUNK, body, 0)

    for k in range(NBUF):
        i = NCHUNK - NBUF + k
        out_copy(i, jax.lax.rem(jnp.int32(i), NBUF)).wait()


@jax.jit
def kernel(image, W1, b1, W2, b2):
    b1r = b1.reshape(1, D_HID)
    b2r = b2.reshape(1, D_OUT)
    return pl.pallas_call(
        _mlp_kernel,
        in_specs=[
            pl.BlockSpec(memory_space=pltpu.MemorySpace.HBM),
            pl.BlockSpec(memory_space=pltpu.MemorySpace.VMEM),
            pl.BlockSpec(memory_space=pltpu.MemorySpace.VMEM),
            pl.BlockSpec(memory_space=pltpu.MemorySpace.VMEM),
            pl.BlockSpec(memory_space=pltpu.MemorySpace.VMEM),
        ],
        out_specs=pl.BlockSpec(memory_space=pltpu.MemorySpace.HBM),
        out_shape=jax.ShapeDtypeStruct((B, D_OUT), jnp.float32),
        scratch_shapes=[
            pltpu.VMEM((NBUF, CH, D_IN), jnp.float32),
            pltpu.VMEM((NBUF, CH, D_OUT), jnp.float32),
            pltpu.SemaphoreType.DMA((NBUF,)),
            pltpu.SemaphoreType.DMA((NBUF,)),
        ],
    )(image, W1, b1r, W2, b2r)


# speedup vs baseline: 1.0532x; 1.0053x over previous
import jax
import jax.numpy as jnp
from jax.experimental import pallas as pl
from jax.experimental.pallas import tpu as pltpu

B = 16384
D_IN = 1024
D_HID = 512
D_OUT = 512

CH = 1024
NCHUNK = B // CH
NBUF = 4


def _mlp_kernel(x_hbm, w1_ref, b1_ref, w2_ref, b2_ref, o_hbm,
                x_bufs, o_bufs, in_sems, out_sems):
    w1 = w1_ref[:]
    b1 = b1_ref[:]
    w2 = w2_ref[:]
    b2 = b2_ref[:]

    def in_copy(i, slot):
        return pltpu.make_async_copy(
            x_hbm.at[pl.ds(i * CH, CH), :], x_bufs.at[slot], in_sems.at[slot])

    def out_copy(i, slot):
        return pltpu.make_async_copy(
            o_bufs.at[slot], o_hbm.at[pl.ds(i * CH, CH), :], out_sems.at[slot])

    for k in range(NBUF):
        in_copy(k, k).start()

    for i in range(NCHUNK):
        slot = i % NBUF
        in_copy(i, slot).wait()

        h = jnp.dot(x_bufs[slot], w1, preferred_element_type=jnp.float32)
        h = jnp.maximum(h + b1, 0.0)
        o = jnp.dot(h, w2, preferred_element_type=jnp.float32) + b2

        if i >= NBUF:
            out_copy(i - NBUF, slot).wait()

        o_bufs[slot] = o
        out_copy(i, slot).start()

        if i + NBUF < NCHUNK:
            in_copy(i + NBUF, slot).start()

    for k in range(NBUF):
        i = NCHUNK - NBUF + k
        out_copy(i, i % NBUF).wait()


@jax.jit
def kernel(image, W1, b1, W2, b2):
    b1r = b1.reshape(1, D_HID)
    b2r = b2.reshape(1, D_OUT)
    return pl.pallas_call(
        _mlp_kernel,
        in_specs=[
            pl.BlockSpec(memory_space=pltpu.MemorySpace.HBM),
            pl.BlockSpec(memory_space=pltpu.MemorySpace.VMEM),
            pl.BlockSpec(memory_space=pltpu.MemorySpace.VMEM),
            pl.BlockSpec(memory_space=pltpu.MemorySpace.VMEM),
            pl.BlockSpec(memory_space=pltpu.MemorySpace.VMEM),
        ],
        out_specs=pl.BlockSpec(memory_space=pltpu.MemorySpace.HBM),
        out_shape=jax.ShapeDtypeStruct((B, D_OUT), jnp.float32),
        scratch_shapes=[
            pltpu.VMEM((NBUF, CH, D_IN), jnp.float32),
            pltpu.VMEM((NBUF, CH, D_OUT), jnp.float32),
            pltpu.SemaphoreType.DMA((NBUF,)),
            pltpu.SemaphoreType.DMA((NBUF,)),
        ],
    )(image, W1, b1r, W2, b2r)
